# Initial kernel scaffold; baseline (speedup 1.0000x reference)
import jax
import jax.numpy as jnp
from jax.experimental import pallas as pl


def kernel(x, edge_index, edge_attr, W_enc, b_enc, W0, b0, W1, b1, Wc1, bc1, Wc2, bc2):
    raise NotImplementedError("write your pallas kernel here")


# speedup vs baseline: 7.9165x; 7.9165x over previous
import functools

import jax
import jax.numpy as jnp
from jax import lax
from jax.experimental import pallas as pl
from jax.experimental.pallas import tpu as pltpu
from jax.experimental.pallas import tpu_sc as plsc


@functools.lru_cache(maxsize=None)
def _make_adj_builder(n_nodes: int, n_edges: int):
    info = plsc.get_sparse_core_info()
    nc, ns, lanes = info.num_cores, info.num_subcores, info.num_lanes
    nw = nc * ns
    cells = n_nodes * n_nodes
    cells_per_sub = cells // ns
    edges_per_tile = n_edges // nw
    chunks = edges_per_tile // lanes

    def body(src_hbm, dst_hbm, w_hbm, out_hbm, src_v, dst_v, w_v, shared, res_v):
        cid = lax.axis_index("c")
        sid = lax.axis_index("s")
        wid = sid * nc + cid
        ebase = wid * edges_per_tile
        pltpu.sync_copy(src_hbm.at[pl.ds(ebase, edges_per_tile)], src_v)
        pltpu.sync_copy(dst_hbm.at[pl.ds(ebase, edges_per_tile)], dst_v)
        pltpu.sync_copy(w_hbm.at[pl.ds(ebase, edges_per_tile)], w_v)

        zero = jnp.zeros((lanes,), jnp.float32)
        for c in range(cells_per_sub // lanes):
            res_v[pl.ds(c * lanes, lanes)] = zero
        pltpu.sync_copy(res_v, shared.at[pl.ds(sid * cells_per_sub,
                                               cells_per_sub)])
        plsc.subcore_barrier()

        for c in range(chunks):
            s = src_v[pl.ds(c * lanes, lanes)]
            d = dst_v[pl.ds(c * lanes, lanes)]
            idx = d * n_nodes + s
            pltpu.sync_copy(w_v.at[pl.ds(c * lanes, lanes)], shared.at[idx],
                            add=True)
        plsc.subcore_barrier()

        pltpu.sync_copy(shared.at[pl.ds(sid * cells_per_sub, cells_per_sub)],
                        res_v)
        pltpu.sync_copy(res_v, out_hbm.at[wid])

    return pl.kernel(
        body,
        out_type=jax.ShapeDtypeStruct((nw, cells_per_sub), jnp.float32),
        mesh=plsc.VectorSubcoreMesh(core_axis_name="c", subcore_axis_name="s"),
        scratch_types=[
            pltpu.VMEM((edges_per_tile,), jnp.int32),
            pltpu.VMEM((edges_per_tile,), jnp.int32),
            pltpu.VMEM((edges_per_tile,), jnp.float32),
            pltpu.VMEM_SHARED((cells,), jnp.float32),
            pltpu.VMEM((cells_per_sub,), jnp.float32),
        ],
    )


def _fwd_body(xT_ref, A0_ref, A1_ref, wenc_ref, benc_ref, W0_ref, b0_ref,
              W1_ref, b1_ref, Wc1_ref, bc1_ref, Wc2_ref, bc2_ref, out_ref):
    n, b = xT_ref.shape
    h_dim = wenc_ref.shape[1]
    xT = xT_ref[...]
    h = xT[:, :, None] * wenc_ref[...][None, :, :] + benc_ref[...][None, None, :]
    A = A0_ref[...] + A1_ref[...]
    for W_ref, b_ref in ((W0_ref, b0_ref), (W1_ref, b1_ref)):
        W = W_ref[...]
        hw = jnp.dot(h.reshape(n * b, h_dim), W,
                     preferred_element_type=jnp.float32) + b_ref[...][None, :]
        hw = jnp.maximum(hw, 0.0)
        msg = jnp.dot(A, hw.reshape(n, b * h_dim),
                      preferred_element_type=jnp.float32)
        h = jnp.maximum(h + msg.reshape(n, b, h_dim), 0.0)
    z = None
    for i in range(n):
        t = jnp.dot(h[i], Wc1_ref[i], preferred_element_type=jnp.float32)
        z = t if z is None else z + t
    z = jnp.maximum(z + bc1_ref[...][None, :], 0.0)
    out_ref[...] = (jnp.dot(z, Wc2_ref[...], preferred_element_type=jnp.float32)
                    + bc2_ref[...][None, :])


def kernel(x, edge_index, edge_attr, W_enc, b_enc, W0, b0, W1, b1, Wc1, bc1,
           Wc2, bc2):
    batch, n_nodes = x.shape
    n_edges = edge_attr.shape[0]
    h_dim = W_enc.shape[1]
    n_out = Wc2.shape[1]

    adj = _make_adj_builder(n_nodes, n_edges)(
        edge_index[0], edge_index[1], edge_attr[:, 0])
    A0 = adj[0::2].reshape(n_nodes, n_nodes)
    A1 = adj[1::2].reshape(n_nodes, n_nodes)

    logits = pl.pallas_call(
        _fwd_body,
        out_shape=jax.ShapeDtypeStruct((batch, n_out), jnp.float32),
    )(x.T, A0, A1, W_enc, b_enc, W0, b0, W1, b1,
      Wc1.reshape(n_nodes, h_dim, h_dim), bc1, Wc2, bc2)
    return logits


# speedup vs baseline: 8.2690x; 1.0445x over previous
import functools

import jax
import jax.numpy as jnp
from jax import lax
from jax.experimental import pallas as pl
from jax.experimental.pallas import tpu as pltpu
from jax.experimental.pallas import tpu_sc as plsc


@functools.lru_cache(maxsize=None)
def _make_adj_builder(n_nodes: int, n_edges: int):
    info = plsc.get_sparse_core_info()
    nc, ns, lanes = info.num_cores, info.num_subcores, info.num_lanes
    nw = nc * ns
    cells = n_nodes * n_nodes
    cells_per_sub = cells // ns
    edges_per_tile = n_edges // nw
    chunks = edges_per_tile // lanes

    def body(ei_hbm, w_hbm, out_hbm, src_v, dst_v, w_v, shared, res_v):
        cid = lax.axis_index("c")
        sid = lax.axis_index("s")
        wid = sid * nc + cid
        ebase = wid * edges_per_tile
        pltpu.sync_copy(ei_hbm.at[0, pl.ds(ebase, edges_per_tile)], src_v)
        pltpu.sync_copy(ei_hbm.at[1, pl.ds(ebase, edges_per_tile)], dst_v)
        pltpu.sync_copy(w_hbm.at[pl.ds(ebase, edges_per_tile)], w_v)

        zero = jnp.zeros((lanes,), jnp.float32)
        for c in range(cells_per_sub // lanes):
            res_v[pl.ds(c * lanes, lanes)] = zero
        pltpu.sync_copy(res_v, shared.at[pl.ds(sid * cells_per_sub,
                                               cells_per_sub)])
        plsc.subcore_barrier()

        for c in range(chunks):
            s = src_v[pl.ds(c * lanes, lanes)]
            d = dst_v[pl.ds(c * lanes, lanes)]
            idx = d * n_nodes + s
            pltpu.sync_copy(w_v.at[pl.ds(c * lanes, lanes)], shared.at[idx],
                            add=True)
        plsc.subcore_barrier()

        pltpu.sync_copy(shared.at[pl.ds(sid * cells_per_sub, cells_per_sub)],
                        res_v)
        pltpu.sync_copy(res_v,
                        out_hbm.at[cid, pl.ds(sid * cells_per_sub,
                                              cells_per_sub)])

    return pl.kernel(
        body,
        out_type=jax.ShapeDtypeStruct((nc, cells), jnp.float32),
        mesh=plsc.VectorSubcoreMesh(core_axis_name="c", subcore_axis_name="s"),
        scratch_types=[
            pltpu.VMEM((edges_per_tile,), jnp.int32),
            pltpu.VMEM((edges_per_tile,), jnp.int32),
            pltpu.VMEM((edges_per_tile,), jnp.float32),
            pltpu.VMEM_SHARED((cells,), jnp.float32),
            pltpu.VMEM((cells_per_sub,), jnp.float32),
        ],
    )


def _fwd_body(x_ref, adj_ref, wenc_ref, benc_ref, W0_ref, b0_ref,
              W1_ref, b1_ref, Wc1_ref, bc1_ref, Wc2_ref, bc2_ref, out_ref):
    b, n = x_ref.shape
    h_dim = wenc_ref.shape[1]
    xT = x_ref[...].T
    h = xT[:, :, None] * wenc_ref[...][None, :, :] + benc_ref[...][None, None, :]
    A = adj_ref[0] + adj_ref[1]
    for W_ref, b_ref in ((W0_ref, b0_ref), (W1_ref, b1_ref)):
        W = W_ref[...]
        hw = jnp.dot(h.reshape(n * b, h_dim), W,
                     preferred_element_type=jnp.float32) + b_ref[...][None, :]
        hw = jnp.maximum(hw, 0.0)
        msg = jnp.dot(A, hw.reshape(n, b * h_dim),
                      preferred_element_type=jnp.float32)
        h = jnp.maximum(h + msg.reshape(n, b, h_dim), 0.0)
    z = None
    for i in range(n):
        t = jnp.dot(h[i], Wc1_ref[i], preferred_element_type=jnp.float32)
        z = t if z is None else z + t
    z = jnp.maximum(z + bc1_ref[...][None, :], 0.0)
    out_ref[...] = (jnp.dot(z, Wc2_ref[...], preferred_element_type=jnp.float32)
                    + bc2_ref[...][None, :])


def kernel(x, edge_index, edge_attr, W_enc, b_enc, W0, b0, W1, b1, Wc1, bc1,
           Wc2, bc2):
    batch, n_nodes = x.shape
    n_edges = edge_attr.shape[0]
    h_dim = W_enc.shape[1]
    n_out = Wc2.shape[1]

    adj = _make_adj_builder(n_nodes, n_edges)(
        edge_index, edge_attr[:, 0]).reshape(2, n_nodes, n_nodes)

    logits = pl.pallas_call(
        _fwd_body,
        out_shape=jax.ShapeDtypeStruct((batch, n_out), jnp.float32),
    )(x, adj, W_enc, b_enc, W0, b0, W1, b1,
      Wc1.reshape(n_nodes, h_dim, h_dim), bc1, Wc2, bc2)
    return logits


# speedup vs baseline: 8.5186x; 1.0302x over previous
import functools

import jax
import jax.numpy as jnp
from jax import lax
from jax.experimental import pallas as pl
from jax.experimental.pallas import tpu as pltpu
from jax.experimental.pallas import tpu_sc as plsc


@functools.lru_cache(maxsize=None)
def _make_adj_builder(n_nodes: int, n_edges: int):
    info = plsc.get_sparse_core_info()
    nc, ns, lanes = info.num_cores, info.num_subcores, info.num_lanes
    nw = nc * ns
    cells = n_nodes * n_nodes
    cells_per_sub = cells // ns
    edges_per_tile = n_edges // nw
    chunks = edges_per_tile // lanes

    def body(ei_hbm, w_hbm, out_hbm, src_v, dst_v, w_v, idx_v, shared, res_v,
             sem_in, sem_w):
        cid = lax.axis_index("c")
        sid = lax.axis_index("s")
        wid = sid * nc + cid
        ebase = wid * edges_per_tile
        cp_src = pltpu.async_copy(ei_hbm.at[0, pl.ds(ebase, edges_per_tile)],
                                  src_v, sem_in)
        cp_dst = pltpu.async_copy(ei_hbm.at[1, pl.ds(ebase, edges_per_tile)],
                                  dst_v, sem_in)
        cp_w = pltpu.async_copy(w_hbm.at[pl.ds(ebase, edges_per_tile)], w_v,
                                sem_w)

        zero = jnp.zeros((lanes,), jnp.float32)
        for c in range(cells_per_sub // lanes):
            res_v[pl.ds(c * lanes, lanes)] = zero
        pltpu.sync_copy(res_v, shared.at[pl.ds(sid * cells_per_sub,
                                               cells_per_sub)])
        cp_src.wait()
        cp_dst.wait()
        for c in range(chunks):
            s = src_v[pl.ds(c * lanes, lanes)]
            d = dst_v[pl.ds(c * lanes, lanes)]
            idx_v[pl.ds(c * lanes, lanes)] = d * n_nodes + s
        cp_w.wait()
        plsc.subcore_barrier()

        pltpu.sync_copy(w_v, shared.at[idx_v], add=True)
        plsc.subcore_barrier()

        pltpu.sync_copy(shared.at[pl.ds(sid * cells_per_sub, cells_per_sub)],
                        res_v)
        pltpu.sync_copy(res_v,
                        out_hbm.at[cid, pl.ds(sid * cells_per_sub,
                                              cells_per_sub)])

    return pl.kernel(
        body,
        out_type=jax.ShapeDtypeStruct((nc, cells), jnp.float32),
        mesh=plsc.VectorSubcoreMesh(core_axis_name="c", subcore_axis_name="s"),
        scratch_types=[
            pltpu.VMEM((edges_per_tile,), jnp.int32),
            pltpu.VMEM((edges_per_tile,), jnp.int32),
            pltpu.VMEM((edges_per_tile,), jnp.float32),
            pltpu.VMEM((edges_per_tile,), jnp.int32),
            pltpu.VMEM_SHARED((cells,), jnp.float32),
            pltpu.VMEM((cells_per_sub,), jnp.float32),
            pltpu.SemaphoreType.DMA,
            pltpu.SemaphoreType.DMA,
        ],
    )


def _fwd_body(x_ref, adj_ref, wenc_ref, benc_ref, W0_ref, b0_ref,
              W1_ref, b1_ref, Wc1_ref, bc1_ref, Wc2_ref, bc2_ref, out_ref):
    b, n = x_ref.shape
    h_dim = wenc_ref.shape[1]
    xT = x_ref[...].T
    h = xT[:, :, None] * wenc_ref[...][None, :, :] + benc_ref[...][None, None, :]
    A = adj_ref[0] + adj_ref[1]
    for W_ref, b_ref in ((W0_ref, b0_ref), (W1_ref, b1_ref)):
        W = W_ref[...]
        hw = jnp.dot(h.reshape(n * b, h_dim), W,
                     preferred_element_type=jnp.float32) + b_ref[...][None, :]
        hw = jnp.maximum(hw, 0.0)
        msg = jnp.dot(A, hw.reshape(n, b * h_dim),
                      preferred_element_type=jnp.float32)
        h = jnp.maximum(h + msg.reshape(n, b, h_dim), 0.0)
    z = None
    for i in range(n):
        t = jnp.dot(h[i], Wc1_ref[i], preferred_element_type=jnp.float32)
        z = t if z is None else z + t
    z = jnp.maximum(z + bc1_ref[...][None, :], 0.0)
    out_ref[...] = (jnp.dot(z, Wc2_ref[...], preferred_element_type=jnp.float32)
                    + bc2_ref[...][None, :])


def kernel(x, edge_index, edge_attr, W_enc, b_enc, W0, b0, W1, b1, Wc1, bc1,
           Wc2, bc2):
    batch, n_nodes = x.shape
    n_edges = edge_attr.shape[0]
    h_dim = W_enc.shape[1]
    n_out = Wc2.shape[1]

    adj = _make_adj_builder(n_nodes, n_edges)(
        edge_index, edge_attr[:, 0]).reshape(2, n_nodes, n_nodes)

    logits = pl.pallas_call(
        _fwd_body,
        out_shape=jax.ShapeDtypeStruct((batch, n_out), jnp.float32),
    )(x, adj, W_enc, b_enc, W0, b0, W1, b1,
      Wc1.reshape(n_nodes, h_dim, h_dim), bc1, Wc2, bc2)
    return logits


# speedup vs baseline: 8.7242x; 1.0241x over previous
import functools

import jax
import jax.numpy as jnp
from jax import lax
from jax.experimental import pallas as pl
from jax.experimental.pallas import tpu as pltpu
from jax.experimental.pallas import tpu_sc as plsc


@functools.lru_cache(maxsize=None)
def _make_adj_builder(n_nodes: int, n_edges: int):
    info = plsc.get_sparse_core_info()
    nc, ns, lanes = info.num_cores, info.num_subcores, info.num_lanes
    nw = nc * ns
    cells = n_nodes * n_nodes
    cells_per_sub = cells // ns
    edges_per_tile = n_edges // nw
    chunks = edges_per_tile // lanes

    def body(ei_hbm, w_hbm, out_hbm, src_v, dst_v, w_v, idx_v, shared, res_v,
             sem_in, sem_w):
        cid = lax.axis_index("c")
        sid = lax.axis_index("s")
        wid = sid * nc + cid
        ebase = wid * edges_per_tile
        cp_src = pltpu.async_copy(ei_hbm.at[0, pl.ds(ebase, edges_per_tile)],
                                  src_v, sem_in)
        cp_dst = pltpu.async_copy(ei_hbm.at[1, pl.ds(ebase, edges_per_tile)],
                                  dst_v, sem_in)
        cp_w = pltpu.async_copy(w_hbm.at[pl.ds(ebase, edges_per_tile)], w_v,
                                sem_w)

        zero = jnp.zeros((lanes,), jnp.float32)
        for c in range(cells_per_sub // lanes):
            res_v[pl.ds(c * lanes, lanes)] = zero
        pltpu.sync_copy(res_v, shared.at[pl.ds(sid * cells_per_sub,
                                               cells_per_sub)])
        cp_src.wait()
        cp_dst.wait()
        for c in range(chunks):
            s = src_v[pl.ds(c * lanes, lanes)]
            d = dst_v[pl.ds(c * lanes, lanes)]
            idx_v[pl.ds(c * lanes, lanes)] = d * n_nodes + s
        cp_w.wait()
        plsc.subcore_barrier()

        pltpu.sync_copy(w_v, shared.at[idx_v], add=True)
        plsc.subcore_barrier()

        pltpu.sync_copy(shared.at[pl.ds(sid * cells_per_sub, cells_per_sub)],
                        res_v)
        pltpu.sync_copy(res_v,
                        out_hbm.at[cid, pl.ds(sid * cells_per_sub,
                                              cells_per_sub)])

    return pl.kernel(
        body,
        out_type=jax.ShapeDtypeStruct((nc, cells), jnp.float32),
        mesh=plsc.VectorSubcoreMesh(core_axis_name="c", subcore_axis_name="s"),
        scratch_types=[
            pltpu.VMEM((edges_per_tile,), jnp.int32),
            pltpu.VMEM((edges_per_tile,), jnp.int32),
            pltpu.VMEM((edges_per_tile,), jnp.float32),
            pltpu.VMEM((edges_per_tile,), jnp.int32),
            pltpu.VMEM_SHARED((cells,), jnp.float32),
            pltpu.VMEM((cells_per_sub,), jnp.float32),
            pltpu.SemaphoreType.DMA,
            pltpu.SemaphoreType.DMA,
        ],
    )


def _lane_linear(h, W, bias, b, h_dim):
    cols = [
        jnp.maximum(
            jnp.dot(h[:, j * h_dim:(j + 1) * h_dim], W,
                    preferred_element_type=jnp.float32) + bias[None, :],
            0.0)
        for j in range(b)
    ]
    return jnp.concatenate(cols, axis=1)


def _pre_body(x_ref, wenc_ref, benc_ref, W0_ref, b0_ref, h0_ref, hw1_ref):
    b, n = x_ref.shape
    h_dim = wenc_ref.shape[1]
    xT = x_ref[...].T
    h3 = (xT[:, :, None] * wenc_ref[...][None, :, :]
          + benc_ref[...][None, None, :])
    h = h3.reshape(n, b * h_dim)
    h0_ref[...] = h
    hw1_ref[...] = _lane_linear(h, W0_ref[...], b0_ref[...], b, h_dim)


def _post_body(adj_ref, h0_ref, hw1_ref, W1_ref, b1_ref, Wc1_ref, bc1_ref,
               Wc2_ref, bc2_ref, out_ref):
    n = adj_ref.shape[1]
    h_dim = W1_ref.shape[0]
    b = h0_ref.shape[1] // h_dim
    A = adj_ref[0] + adj_ref[1]
    msg1 = jnp.dot(A, hw1_ref[...], preferred_element_type=jnp.float32)
    h = jnp.maximum(h0_ref[...] + msg1, 0.0)
    hw2 = _lane_linear(h, W1_ref[...], b1_ref[...], b, h_dim)
    msg2 = jnp.dot(A, hw2, preferred_element_type=jnp.float32)
    h = jnp.maximum(h + msg2, 0.0)
    hf = jnp.transpose(h.reshape(n, b, h_dim), (1, 0, 2)).reshape(b, n * h_dim)
    z = jnp.dot(hf, Wc1_ref[...].reshape(n * h_dim, h_dim),
                preferred_element_type=jnp.float32)
    z = jnp.maximum(z + bc1_ref[...][None, :], 0.0)
    out_ref[...] = (jnp.dot(z, Wc2_ref[...], preferred_element_type=jnp.float32)
                    + bc2_ref[...][None, :])


def kernel(x, edge_index, edge_attr, W_enc, b_enc, W0, b0, W1, b1, Wc1, bc1,
           Wc2, bc2):
    batch, n_nodes = x.shape
    n_edges = edge_attr.shape[0]
    h_dim = W_enc.shape[1]
    n_out = Wc2.shape[1]

    adj = _make_adj_builder(n_nodes, n_edges)(
        edge_index, edge_attr[:, 0]).reshape(2, n_nodes, n_nodes)

    h0, hw1 = pl.pallas_call(
        _pre_body,
        out_shape=[
            jax.ShapeDtypeStruct((n_nodes, batch * h_dim), jnp.float32),
            jax.ShapeDtypeStruct((n_nodes, batch * h_dim), jnp.float32),
        ],
    )(x, W_enc, b_enc, W0, b0)

    logits = pl.pallas_call(
        _post_body,
        out_shape=jax.ShapeDtypeStruct((batch, n_out), jnp.float32),
    )(adj, h0, hw1, W1, b1, Wc1.reshape(n_nodes, h_dim, h_dim), bc1, Wc2, bc2)
    return logits


# speedup vs baseline: 9.1481x; 1.0486x over previous
import functools

import jax
import jax.numpy as jnp
from jax import lax
from jax.experimental import pallas as pl
from jax.experimental.pallas import tpu as pltpu
from jax.experimental.pallas import tpu_sc as plsc


@functools.lru_cache(maxsize=None)
def _make_adj_builder(n_nodes: int, n_edges: int):
    info = plsc.get_sparse_core_info()
    nc, ns, lanes = info.num_cores, info.num_subcores, info.num_lanes
    nw = nc * ns
    cells = n_nodes * n_nodes
    cells_per_sub = cells // ns
    edges_per_tile = n_edges // nw
    chunks = edges_per_tile // lanes

    def body(ei_hbm, w_hbm, out_hbm, src_v, dst_v, w_v, idx_v, shared, res_v,
             sem_in, sem_w):
        cid = lax.axis_index("c")
        sid = lax.axis_index("s")
        wid = sid * nc + cid
        ebase = wid * edges_per_tile
        cp_src = pltpu.async_copy(ei_hbm.at[0, pl.ds(ebase, edges_per_tile)],
                                  src_v, sem_in)
        cp_dst = pltpu.async_copy(ei_hbm.at[1, pl.ds(ebase, edges_per_tile)],
                                  dst_v, sem_in)
        cp_w = pltpu.async_copy(w_hbm.at[pl.ds(ebase, edges_per_tile)], w_v,
                                sem_w)

        zero = jnp.zeros((lanes,), jnp.float32)
        for c in range(cells_per_sub // lanes):
            res_v[pl.ds(c * lanes, lanes)] = zero
        pltpu.sync_copy(res_v, shared.at[pl.ds(sid * cells_per_sub,
                                               cells_per_sub)])
        cp_src.wait()
        cp_dst.wait()
        for c in range(chunks):
            s = src_v[pl.ds(c * lanes, lanes)]
            d = dst_v[pl.ds(c * lanes, lanes)]
            idx_v[pl.ds(c * lanes, lanes)] = d * n_nodes + s
        cp_w.wait()
        plsc.subcore_barrier()

        pltpu.sync_copy(w_v, shared.at[idx_v], add=True)
        plsc.subcore_barrier()

        pltpu.sync_copy(shared.at[pl.ds(sid * cells_per_sub, cells_per_sub)],
                        out_hbm.at[cid, pl.ds(sid * cells_per_sub,
                                              cells_per_sub)])

    return pl.kernel(
        body,
        out_type=jax.ShapeDtypeStruct((nc, cells), jnp.float32),
        mesh=plsc.VectorSubcoreMesh(core_axis_name="c", subcore_axis_name="s"),
        scratch_types=[
            pltpu.VMEM((edges_per_tile,), jnp.int32),
            pltpu.VMEM((edges_per_tile,), jnp.int32),
            pltpu.VMEM((edges_per_tile,), jnp.float32),
            pltpu.VMEM((edges_per_tile,), jnp.int32),
            pltpu.VMEM_SHARED((cells,), jnp.float32),
            pltpu.VMEM((cells_per_sub,), jnp.float32),
            pltpu.SemaphoreType.DMA,
            pltpu.SemaphoreType.DMA,
        ],
    )


def _lane_linear(h, W, bias, b, h_dim):
    cols = [
        jnp.maximum(
            jnp.dot(h[:, j * h_dim:(j + 1) * h_dim], W,
                    preferred_element_type=jnp.float32) + bias[None, :],
            0.0)
        for j in range(b)
    ]
    return jnp.concatenate(cols, axis=1)


def _pre_body(x_ref, wenc_ref, benc_ref, W0_ref, b0_ref, h0_ref, hw1_ref):
    b, n = x_ref.shape
    h_dim = wenc_ref.shape[1]
    xT = x_ref[...].T
    h3 = (xT[:, :, None] * wenc_ref[...][None, :, :]
          + benc_ref[...][None, None, :])
    h = h3.reshape(n, b * h_dim)
    h0_ref[...] = h
    hw1_ref[...] = _lane_linear(h, W0_ref[...], b0_ref[...], b, h_dim)


def _post_body(adj_ref, h0_ref, hw1_ref, W1_ref, b1_ref, Wc1_ref, bc1_ref,
               Wc2_ref, bc2_ref, out_ref):
    h_dim = W1_ref.shape[0]
    b = h0_ref.shape[1] // h_dim
    n = h0_ref.shape[0]
    a_flat = adj_ref[0] + adj_ref[1]
    A = jnp.concatenate(
        [a_flat[j * n:(j + 1) * n][None, :] for j in range(n)], axis=0)
    msg1 = jnp.dot(A, hw1_ref[...], preferred_element_type=jnp.float32)
    h = jnp.maximum(h0_ref[...] + msg1, 0.0)
    hw2 = _lane_linear(h, W1_ref[...], b1_ref[...], b, h_dim)
    msg2 = jnp.dot(A, hw2, preferred_element_type=jnp.float32)
    h = jnp.maximum(h + msg2, 0.0)
    hf = jnp.transpose(h.reshape(n, b, h_dim), (1, 0, 2)).reshape(b, n * h_dim)
    z = jnp.dot(hf, Wc1_ref[...].reshape(n * h_dim, h_dim),
                preferred_element_type=jnp.float32)
    z = jnp.maximum(z + bc1_ref[...][None, :], 0.0)
    out_ref[...] = (jnp.dot(z, Wc2_ref[...], preferred_element_type=jnp.float32)
                    + bc2_ref[...][None, :])


def kernel(x, edge_index, edge_attr, W_enc, b_enc, W0, b0, W1, b1, Wc1, bc1,
           Wc2, bc2):
    batch, n_nodes = x.shape
    n_edges = edge_attr.shape[0]
    h_dim = W_enc.shape[1]
    n_out = Wc2.shape[1]

    adj = _make_adj_builder(n_nodes, n_edges)(edge_index, edge_attr[:, 0])

    h0, hw1 = pl.pallas_call(
        _pre_body,
        out_shape=[
            jax.ShapeDtypeStruct((n_nodes, batch * h_dim), jnp.float32),
            jax.ShapeDtypeStruct((n_nodes, batch * h_dim), jnp.float32),
        ],
    )(x, W_enc, b_enc, W0, b0)

    logits = pl.pallas_call(
        _post_body,
        out_shape=jax.ShapeDtypeStruct((batch, n_out), jnp.float32),
    )(adj, h0, hw1, W1, b1, Wc1.reshape(n_nodes, h_dim, h_dim), bc1, Wc2, bc2)
    return logits
